# baseline (speedup 1.0000x reference)
import functools

import jax
import jax.numpy as jnp
from jax import lax
from jax.experimental import pallas as pl
from jax.experimental.pallas import tpu as pltpu
from jax.experimental.pallas import tpu_sc as plsc

M = 1000000
D = 128
B = 16384
NC = 2
NS = 16
L = 16
NW = NC * NS
BPW = B // NW
GCHUNK = 64
NG = BPW // GCHUNK
NBUF = 2


def _make_sc_kernel():
    mesh = plsc.VectorSubcoreMesh(core_axis_name="c", subcore_axis_name="s")

    @functools.partial(
        pl.kernel,
        mesh=mesh,
        out_type=jax.ShapeDtypeStruct((B, D), jnp.float32),
        compiler_params=pltpu.CompilerParams(needs_layout_passes=False),
        scratch_types=[
            pltpu.VMEM((NG, GCHUNK), jnp.int32),
            pltpu.VMEM((BPW,), jnp.float32),
            pltpu.VMEM((D,), jnp.float32),
            pltpu.VMEM((D,), jnp.float32),
            pltpu.VMEM((BPW, D), jnp.float32),
            pltpu.VMEM((NBUF, GCHUNK, D), jnp.float32),
            pltpu.SemaphoreType.DMA,
            pltpu.SemaphoreType.DMA,
            [pltpu.SemaphoreType.DMA] * NG,
            [pltpu.SemaphoreType.DMA] * NBUF,
        ],
    )
    def sc_kernel(mem_hbm, idx_hbm, td_hbm, w_hbm, b_hbm, out_hbm,
                  idx_v, td_v, w_v, b_v, rows_v, obuf_v,
                  sem_idx, sem_stage, sems_g, sems_out):
        wid = lax.axis_index("s") * NC + lax.axis_index("c")
        base = wid * BPW

        c_idx = pltpu.async_copy(idx_hbm.at[wid], idx_v, sem_idx)
        c_td = pltpu.async_copy(td_hbm.at[pl.ds(base, BPW)], td_v, sem_stage)
        c_w = pltpu.async_copy(w_hbm, w_v, sem_stage)
        c_b = pltpu.async_copy(b_hbm, b_v, sem_stage)

        c_idx.wait()
        gathers = []
        for g in range(NG):
            gathers.append(pltpu.async_copy(
                mem_hbm.at[idx_v.at[g]],
                rows_v.at[pl.ds(g * GCHUNK, GCHUNK)],
                sems_g[g],
            ))
        c_td.wait()
        c_w.wait()
        c_b.wait()

        w_chunks = [w_v[pl.ds(c * L, L)] for c in range(D // L)]
        b_chunks = [b_v[pl.ds(c * L, L)] + 1.0 for c in range(D // L)]

        stores = [None] * NG
        for g in range(NG):
            s = g % NBUF
            if g >= NBUF:
                stores[g - NBUF].wait()
            gathers[g].wait()
            off = g * GCHUNK

            def row_body(i, carry):
                tdv = plsc.load_gather(
                    td_v, [jnp.full((L,), off + i, jnp.int32)])
                for c in range(D // L):
                    sl = pl.ds(c * L, L)
                    obuf_v[s, i, sl] = (
                        rows_v[off + i, sl] * (tdv * w_chunks[c] + b_chunks[c]))
                return carry

            lax.fori_loop(0, GCHUNK, row_body, 0)

            stores[g] = pltpu.async_copy(
                obuf_v.at[s],
                out_hbm.at[pl.ds(base + off, GCHUNK)],
                sems_out[s],
            )
        for g in range(NG - NBUF, NG):
            stores[g].wait()

    return sc_kernel


_sc_kernel = _make_sc_kernel()


def kernel(memory, source_nodes, timestamps, time_diffs, W, b):
    del timestamps
    idx = source_nodes.astype(jnp.int32).reshape(NW, NG, GCHUNK)
    w_flat = W.reshape(D)
    return _sc_kernel(memory, idx, time_diffs.astype(jnp.float32), w_flat, b)


# speedup vs baseline: 1.1106x; 1.1106x over previous
import functools

import jax
import jax.numpy as jnp
from jax import lax
from jax.experimental import pallas as pl
from jax.experimental.pallas import tpu as pltpu
from jax.experimental.pallas import tpu_sc as plsc

M = 1000000
D = 128
B = 16384
NC = 2
NS = 16
L = 16
NW = NC * NS
BPW = B // NW
CHUNKS = (128, 128, 128, 96, 32)
NG = len(CHUNKS)
RUNROLL = 2


def _make_sc_kernel():
    mesh = plsc.VectorSubcoreMesh(core_axis_name="c", subcore_axis_name="s")

    @functools.partial(
        pl.kernel,
        mesh=mesh,
        out_type=jax.ShapeDtypeStruct((B, D), jnp.float32),
        compiler_params=pltpu.CompilerParams(needs_layout_passes=False),
        scratch_types=[
            pltpu.VMEM((BPW,), jnp.int32),
            pltpu.VMEM((BPW,), jnp.float32),
            pltpu.VMEM((D,), jnp.float32),
            pltpu.VMEM((D,), jnp.float32),
            pltpu.VMEM((BPW, D), jnp.float32),
            pltpu.SemaphoreType.DMA,
            pltpu.SemaphoreType.DMA,
            [pltpu.SemaphoreType.DMA] * NG,
            pltpu.SemaphoreType.DMA,
        ],
    )
    def sc_kernel(mem_hbm, idx_hbm, td_hbm, w_hbm, b_hbm, out_hbm,
                  idx_v, td_v, w_v, b_v, rows_v,
                  sem_idx, sem_stage, sems_g, sem_out):
        wid = lax.axis_index("s") * NC + lax.axis_index("c")
        base = wid * BPW

        c_idx = pltpu.async_copy(idx_hbm.at[wid], idx_v, sem_idx)
        c_td = pltpu.async_copy(td_hbm.at[pl.ds(base, BPW)], td_v, sem_stage)
        c_w = pltpu.async_copy(w_hbm, w_v, sem_stage)
        c_b = pltpu.async_copy(b_hbm, b_v, sem_stage)

        c_idx.wait()
        gathers = []
        off = 0
        for g, sz in enumerate(CHUNKS):
            gathers.append(pltpu.async_copy(
                mem_hbm.at[idx_v.at[pl.ds(off, sz)]],
                rows_v.at[pl.ds(off, sz)],
                sems_g[g],
            ))
            off += sz
        c_td.wait()
        c_w.wait()
        c_b.wait()

        w_chunks = [w_v[pl.ds(c * L, L)] for c in range(D // L)]
        b_chunks = [b_v[pl.ds(c * L, L)] + 1.0 for c in range(D // L)]

        def scale_row(i):
            tdv = plsc.load_gather(td_v, [jnp.full((L,), i, jnp.int32)])
            row = [rows_v[i, pl.ds(c * L, L)] for c in range(D // L)]
            out = [row[c] * (tdv * w_chunks[c] + b_chunks[c])
                   for c in range(D // L)]
            for c in range(D // L):
                rows_v[i, pl.ds(c * L, L)] = out[c]

        stores = []
        off = 0
        for g, sz in enumerate(CHUNKS):
            gathers[g].wait()
            chunk_off = off

            def row_body(k, carry, chunk_off=chunk_off):
                i = chunk_off + k * RUNROLL
                for r in range(RUNROLL):
                    scale_row(i + r)
                return carry

            lax.fori_loop(0, sz // RUNROLL, row_body, 0)

            stores.append(pltpu.async_copy(
                rows_v.at[pl.ds(off, sz)],
                out_hbm.at[pl.ds(base + off, sz)],
                sem_out,
            ))
            off += sz
        for s in stores:
            s.wait()

    return sc_kernel


_sc_kernel = _make_sc_kernel()


def kernel(memory, source_nodes, timestamps, time_diffs, W, b):
    del timestamps
    idx = source_nodes.astype(jnp.int32).reshape(NW, BPW)
    w_flat = W.reshape(D)
    return _sc_kernel(memory, idx, time_diffs.astype(jnp.float32), w_flat, b)


# speedup vs baseline: 1.1126x; 1.0018x over previous
import functools

import jax
import jax.numpy as jnp
from jax import lax
from jax.experimental import pallas as pl
from jax.experimental.pallas import tpu as pltpu
from jax.experimental.pallas import tpu_sc as plsc

M = 1000000
D = 128
B = 16384
NC = 2
NS = 16
L = 16
NW = NC * NS
BPW = B // NW
CHUNKS = (128, 128, 128, 96, 32)
NG = len(CHUNKS)
RUNROLL = 2
PRELUDE = BPW + BPW + D + D
TD_OFF = BPW
W_OFF = 2 * BPW
B_OFF = 2 * BPW + D


def _make_sc_kernel():
    mesh = plsc.VectorSubcoreMesh(core_axis_name="c", subcore_axis_name="s")

    @functools.partial(
        pl.kernel,
        mesh=mesh,
        out_type=jax.ShapeDtypeStruct((B, D), jnp.float32),
        compiler_params=pltpu.CompilerParams(needs_layout_passes=False),
        scratch_types=[
            pltpu.VMEM((PRELUDE,), jnp.int32),
            pltpu.VMEM((BPW, D), jnp.float32),
            pltpu.SemaphoreType.DMA,
            [pltpu.SemaphoreType.DMA] * NG,
            pltpu.SemaphoreType.DMA,
        ],
    )
    def sc_kernel(mem_hbm, pre_hbm, out_hbm,
                  pre_v, rows_v, sem_pre, sems_g, sem_out):
        wid = lax.axis_index("s") * NC + lax.axis_index("c")
        base = wid * BPW

        pltpu.async_copy(pre_hbm.at[wid], pre_v, sem_pre).wait()

        gathers = []
        off = 0
        for g, sz in enumerate(CHUNKS):
            gathers.append(pltpu.async_copy(
                mem_hbm.at[pre_v.at[pl.ds(off, sz)]],
                rows_v.at[pl.ds(off, sz)],
                sems_g[g],
            ))
            off += sz

        w_chunks = [
            plsc.bitcast(pre_v[pl.ds(W_OFF + c * L, L)], jnp.float32)
            for c in range(D // L)
        ]
        b_chunks = [
            plsc.bitcast(pre_v[pl.ds(B_OFF + c * L, L)], jnp.float32) + 1.0
            for c in range(D // L)
        ]

        def scale_row(i):
            tdv = plsc.bitcast(
                plsc.load_gather(pre_v, [jnp.full((L,), TD_OFF + i, jnp.int32)]),
                jnp.float32)
            row = [rows_v[i, pl.ds(c * L, L)] for c in range(D // L)]
            out = [row[c] * (tdv * w_chunks[c] + b_chunks[c])
                   for c in range(D // L)]
            for c in range(D // L):
                rows_v[i, pl.ds(c * L, L)] = out[c]

        stores = []
        off = 0
        for g, sz in enumerate(CHUNKS):
            gathers[g].wait()
            chunk_off = off

            def row_body(k, carry, chunk_off=chunk_off):
                i = chunk_off + k * RUNROLL
                for r in range(RUNROLL):
                    scale_row(i + r)
                return carry

            lax.fori_loop(0, sz // RUNROLL, row_body, 0)

            stores.append(pltpu.async_copy(
                rows_v.at[pl.ds(off, sz)],
                out_hbm.at[pl.ds(base + off, sz)],
                sem_out,
            ))
            off += sz
        for s in stores:
            s.wait()

    return sc_kernel


_sc_kernel = _make_sc_kernel()


def kernel(memory, source_nodes, timestamps, time_diffs, W, b):
    del timestamps
    idx = source_nodes.astype(jnp.int32).reshape(NW, BPW)
    td = time_diffs.astype(jnp.float32).reshape(NW, BPW)
    wb = jnp.concatenate([W.reshape(D), b]).astype(jnp.float32)
    wb32 = jnp.broadcast_to(wb[None, :], (NW, 2 * D))
    pre = jnp.concatenate(
        [idx, jax.lax.bitcast_convert_type(td, jnp.int32),
         jax.lax.bitcast_convert_type(wb32, jnp.int32)], axis=1)
    return _sc_kernel(memory, pre)


# speedup vs baseline: 1.1374x; 1.0223x over previous
import functools

import jax
import jax.numpy as jnp
from jax import lax
from jax.experimental import pallas as pl
from jax.experimental.pallas import tpu as pltpu
from jax.experimental.pallas import tpu_sc as plsc

M = 1000000
D = 128
B = 16384
NC = 2
NS = 16
L = 16
NW = NC * NS
BPW = B // NW
CHUNKS = (128, 128, 128, 96, 32)
NG = len(CHUNKS)
RUNROLL = 1
PRELUDE = BPW + BPW + D + D
TD_OFF = BPW
W_OFF = 2 * BPW
B_OFF = 2 * BPW + D


def _make_sc_kernel():
    mesh = plsc.VectorSubcoreMesh(core_axis_name="c", subcore_axis_name="s")

    @functools.partial(
        pl.kernel,
        mesh=mesh,
        out_type=jax.ShapeDtypeStruct((B, D), jnp.float32),
        compiler_params=pltpu.CompilerParams(needs_layout_passes=False),
        scratch_types=[
            pltpu.VMEM((PRELUDE,), jnp.int32),
            pltpu.VMEM((BPW, D), jnp.float32),
            pltpu.SemaphoreType.DMA,
            pltpu.SemaphoreType.DMA,
            [pltpu.SemaphoreType.DMA] * NG,
            pltpu.SemaphoreType.DMA,
        ],
    )
    def sc_kernel(mem_hbm, pre_hbm, out_hbm,
                  pre_v, rows_v, sem_pre0, sem_pre1, sems_g, sem_out):
        wid = lax.axis_index("s") * NC + lax.axis_index("c")
        base = wid * BPW

        sz0 = CHUNKS[0]
        c_head = pltpu.async_copy(
            pre_hbm.at[wid, pl.ds(0, sz0)], pre_v.at[pl.ds(0, sz0)], sem_pre0)
        c_rest = pltpu.async_copy(
            pre_hbm.at[wid, pl.ds(sz0, PRELUDE - sz0)],
            pre_v.at[pl.ds(sz0, PRELUDE - sz0)], sem_pre1)

        c_head.wait()
        gathers = [pltpu.async_copy(
            mem_hbm.at[pre_v.at[pl.ds(0, sz0)]],
            rows_v.at[pl.ds(0, sz0)],
            sems_g[0],
        )]
        c_rest.wait()
        off = sz0
        for g, sz in enumerate(CHUNKS[1:], start=1):
            gathers.append(pltpu.async_copy(
                mem_hbm.at[pre_v.at[pl.ds(off, sz)]],
                rows_v.at[pl.ds(off, sz)],
                sems_g[g],
            ))
            off += sz

        w_chunks = [
            plsc.bitcast(pre_v[pl.ds(W_OFF + c * L, L)], jnp.float32)
            for c in range(D // L)
        ]
        b_chunks = [
            plsc.bitcast(pre_v[pl.ds(B_OFF + c * L, L)], jnp.float32) + 1.0
            for c in range(D // L)
        ]

        def scale_row(i):
            tdv = plsc.bitcast(
                plsc.load_gather(pre_v, [jnp.full((L,), TD_OFF + i, jnp.int32)]),
                jnp.float32)
            row = [rows_v[i, pl.ds(c * L, L)] for c in range(D // L)]
            out = [row[c] * (tdv * w_chunks[c] + b_chunks[c])
                   for c in range(D // L)]
            for c in range(D // L):
                rows_v[i, pl.ds(c * L, L)] = out[c]

        stores = []
        off = 0
        for g, sz in enumerate(CHUNKS):
            gathers[g].wait()
            chunk_off = off

            def row_body(k, carry, chunk_off=chunk_off):
                i = chunk_off + k * RUNROLL
                for r in range(RUNROLL):
                    scale_row(i + r)
                return carry

            lax.fori_loop(0, sz // RUNROLL, row_body, 0)

            stores.append(pltpu.async_copy(
                rows_v.at[pl.ds(off, sz)],
                out_hbm.at[pl.ds(base + off, sz)],
                sem_out,
            ))
            off += sz
        for s in stores:
            s.wait()

    return sc_kernel


_sc_kernel = _make_sc_kernel()


def kernel(memory, source_nodes, timestamps, time_diffs, W, b):
    del timestamps
    idx = source_nodes.astype(jnp.int32).reshape(NW, BPW)
    td = time_diffs.astype(jnp.float32).reshape(NW, BPW)
    wb = jnp.concatenate([W.reshape(D), b]).astype(jnp.float32)
    wb32 = jnp.broadcast_to(wb[None, :], (NW, 2 * D))
    pre = jnp.concatenate(
        [idx, jax.lax.bitcast_convert_type(td, jnp.int32),
         jax.lax.bitcast_convert_type(wb32, jnp.int32)], axis=1)
    return _sc_kernel(memory, pre)
